# baseline (speedup 1.0000x reference)
import functools

import jax
import jax.numpy as jnp
import numpy as np
from jax import lax
from jax.experimental import pallas as pl
from jax.experimental.pallas import tpu as pltpu
from jax.experimental.pallas import tpu_sc as plsc

N_UNIQ = 100000
STATE_SIZE = 128
HALF = 8192
NUM_LEFT = N_UNIQ - HALF

V = 1024
C = 200
B = 512
N_PAD = B * C
PAD_VAL = 1023

NW = 32
LANES = 16
LROW = 128
CPW = LANES
EPT = CPW * C
KB = (2 * HALF) // NW


def _np_threefry2x32(k1, k2, x0, x1):
    def rotl(x, d):
        return ((x << np.uint32(d)) | (x >> np.uint32(32 - d))).astype(np.uint32)
    ks = [np.uint32(k1), np.uint32(k2),
          np.uint32(k1) ^ np.uint32(k2) ^ np.uint32(0x1BD11BDA)]
    rots = [(13, 15, 26, 6), (17, 29, 16, 24)]
    x0 = (x0 + ks[0]).astype(np.uint32)
    x1 = (x1 + ks[1]).astype(np.uint32)
    for i in range(5):
        for r in rots[i % 2]:
            x0 = (x0 + x1).astype(np.uint32)
            x1 = rotl(x1, r)
            x1 = x0 ^ x1
        x0 = (x0 + ks[(i + 1) % 3]).astype(np.uint32)
        x1 = (x1 + ks[(i + 2) % 3] + np.uint32(i + 1)).astype(np.uint32)
    return x0, x1


def _np_permutation(seed, n):
    key = np.array([np.uint32(np.uint64(seed) >> np.uint64(32)),
                    np.uint32(np.uint64(seed) & np.uint64(0xFFFFFFFF))])
    x = np.arange(n, dtype=np.int32)
    num_rounds = int(np.ceil(3 * np.log(max(1, n)) / np.log(2**32 - 1)))
    for _ in range(num_rounds):
        b1, b2 = _np_threefry2x32(key[0], key[1],
                                  np.zeros(2, np.uint32),
                                  np.arange(2, dtype=np.uint32))
        key, subkey = np.stack([b1, b2], 1)
        s1, s2 = _np_threefry2x32(subkey[0], subkey[1],
                                  np.zeros(n, np.uint32),
                                  np.arange(n, dtype=np.uint32))
        x = x[np.argsort(s1 ^ s2, kind="stable")]
    return x


_PERM = _np_permutation(12345, NUM_LEFT)[:HALF]
_POS_TAB = np.concatenate([NUM_LEFT + np.arange(HALF, dtype=np.int32),
                           _PERM.astype(np.int32)])
_SELBITS = np.zeros(N_PAD // 32, np.int32)
np.bitwise_or.at(_SELBITS, _POS_TAB >> 5,
                 np.int32(1) << (_POS_TAB & 31))


def _sc_histrank_body(cnt_hbm, hist_out, rin_out, key_out,
                      cnt_v, hist_v, rin_v, key_v):
    wid = lax.axis_index("s") * 2 + lax.axis_index("c")
    base = wid * EPT
    pltpu.sync_copy(cnt_hbm.at[pl.ds(base, EPT)], cnt_v)
    iota16 = lax.broadcasted_iota(jnp.int32, (LANES,), 0)
    zeros16 = jnp.zeros((LANES,), jnp.int32)
    ones16 = jnp.ones((LANES,), jnp.int32)
    kbase = wid * CPW * V

    def zero_step(i, _):
        hist_v[pl.ds(i * LANES, LANES)] = zeros16
        return 0

    lax.fori_loop(0, CPW * V // LANES, zero_step, 0)

    def elem_step(e, _):
        idx16 = iota16 * C + e
        c16 = plsc.load_gather(cnt_v, [idx16])
        h_idx = iota16 * V + c16
        old = plsc.load_gather(hist_v, [h_idx])
        plsc.store_scatter(rin_v, [idx16], old)
        plsc.store_scatter(key_v, [idx16], h_idx + kbase)
        plsc.addupdate_scatter(hist_v, [h_idx], ones16)
        return 0

    lax.fori_loop(0, C, elem_step, 0)
    pltpu.sync_copy(hist_v, hist_out.at[pl.ds(wid * CPW * V, CPW * V)])
    pltpu.sync_copy(rin_v, rin_out.at[pl.ds(base, EPT)])
    pltpu.sync_copy(key_v, key_out.at[pl.ds(base, EPT)])


def _tc_offs_body(hist_ref, out_ref):
    hist = hist_ref[...].astype(jnp.float32)
    tri = (lax.broadcasted_iota(jnp.int32, (B, B), 1)
           < lax.broadcasted_iota(jnp.int32, (B, B), 0)).astype(jnp.float32)
    colcum = jnp.dot(tri, hist, preferred_element_type=jnp.float32)
    total = jnp.sum(hist, axis=0, keepdims=True)
    mv = (lax.broadcasted_iota(jnp.int32, (V, V), 0)
          < lax.broadcasted_iota(jnp.int32, (V, V), 1)).astype(jnp.float32)
    glob = jnp.dot(total, mv, preferred_element_type=jnp.float32)
    out_ref[...] = (colcum + glob).astype(jnp.int32)


def _sc_scatter_body(key_hbm, rin_hbm, choff_hbm, selb_hbm, sorted_out,
                     key_v, rin_v, choff_v, selb_v, ids_v, pos_v, sem):
    wid = lax.axis_index("s") * 2 + lax.axis_index("c")
    base = wid * EPT
    pltpu.sync_copy(key_hbm.at[pl.ds(base, EPT)], key_v)
    pltpu.sync_copy(rin_hbm.at[pl.ds(base, EPT)], rin_v)
    pltpu.sync_copy(choff_hbm.at[pl.ds(wid * CPW * V, CPW * V)], choff_v)
    pltpu.sync_copy(selb_hbm, selb_v)
    iota16 = lax.broadcasted_iota(jnp.int32, (LANES,), 0)
    kbase = wid * CPW * V
    gpr = LROW // LANES
    trash16 = N_PAD + wid * LROW + iota16

    def pos_step(g, _):
        k16 = key_v[pl.ds(g * LANES, LANES)] - kbase
        off16 = plsc.load_gather(choff_v, [k16])
        p16 = off16 + rin_v[pl.ds(g * LANES, LANES)]
        w16 = plsc.load_gather(selb_v, [lax.shift_right_logical(p16, 5)])
        bit = lax.shift_right_logical(w16, p16 & 31) & 1
        pos_v[g // gpr, pl.ds((g % gpr) * LANES, LANES)] = jnp.where(
            bit != 0, p16, trash16)
        ids_v[pl.ds(g * LANES, LANES)] = base + g * LANES + iota16
        return 0

    lax.fori_loop(0, EPT // LANES, pos_step, 0)
    cps = [pltpu.async_copy(ids_v.at[pl.ds(j * LROW, LROW)],
                            sorted_out.at[pos_v.at[j]], sem)
           for j in range(EPT // LROW)]
    for cp in cps:
        cp.wait()


def _sc_gather_body(ptab_hbm, sorted_hbm, cnt_hbm, samp_hbm,
                    samp_out, cnt_out, ptab_v, id_v, cntg_v, rows_v, sem):
    wid = lax.axis_index("s") * 2 + lax.axis_index("c")
    base = wid * KB
    pltpu.sync_copy(ptab_hbm.at[pl.ds(base, KB)], ptab_v)
    cps = [pltpu.async_copy(sorted_hbm.at[ptab_v.at[pl.ds(j * LROW, LROW)]],
                            id_v.at[pl.ds(j * LROW, LROW)], sem)
           for j in range(KB // LROW)]
    for cp in cps:
        cp.wait()
    cps = [pltpu.async_copy(cnt_hbm.at[id_v.at[pl.ds(j * LROW, LROW)]],
                            cntg_v.at[pl.ds(j * LROW, LROW)], sem)
           for j in range(KB // LROW)]
    cps += [pltpu.async_copy(samp_hbm.at[id_v.at[pl.ds(j * LROW, LROW)]],
                             rows_v.at[pl.ds(j * LROW, LROW)], sem)
            for j in range(KB // LROW)]
    for cp in cps:
        cp.wait()
    pltpu.sync_copy(cntg_v, cnt_out.at[pl.ds(base, KB)])
    pltpu.sync_copy(rows_v, samp_out.at[pl.ds(base, KB)])


def kernel(uniq_samples, uniq_count):
    i32 = jnp.int32
    cpad = jnp.concatenate(
        [uniq_count, jnp.full((N_PAD - N_UNIQ,), PAD_VAL, dtype=i32)])

    mesh = plsc.VectorSubcoreMesh(
        core_axis_name="c", subcore_axis_name="s",
        num_cores=2, num_subcores=16)

    sc_histrank = functools.partial(
        pl.kernel,
        out_type=(
            jax.ShapeDtypeStruct((B * V,), i32),
            jax.ShapeDtypeStruct((N_PAD,), i32),
            jax.ShapeDtypeStruct((N_PAD,), i32),
        ),
        mesh=mesh,
        scratch_types=[pltpu.VMEM((EPT,), i32),
                       pltpu.VMEM((CPW * V,), i32),
                       pltpu.VMEM((EPT,), i32),
                       pltpu.VMEM((EPT,), i32)],
        compiler_params=pltpu.CompilerParams(needs_layout_passes=False),
    )(_sc_histrank_body)

    hist, rin, key = sc_histrank(cpad)

    choff = pl.pallas_call(
        _tc_offs_body,
        out_shape=jax.ShapeDtypeStruct((B, V), i32),
    )(hist.reshape(B, V))

    sc_scatter = functools.partial(
        pl.kernel,
        out_type=jax.ShapeDtypeStruct((N_PAD + NW * LROW,), i32),
        mesh=mesh,
        scratch_types=[
            pltpu.VMEM((EPT,), i32),
            pltpu.VMEM((EPT,), i32),
            pltpu.VMEM((CPW * V,), i32),
            pltpu.VMEM((N_PAD // 32,), i32),
            pltpu.VMEM((EPT,), i32),
            pltpu.VMEM((EPT // LROW, LROW), i32),
            pltpu.SemaphoreType.DMA,
        ],
        compiler_params=pltpu.CompilerParams(needs_layout_passes=False),
    )(_sc_scatter_body)

    sorted_ids = sc_scatter(key, rin, choff.reshape(-1), jnp.asarray(_SELBITS))

    sc_gather = functools.partial(
        pl.kernel,
        out_type=(
            jax.ShapeDtypeStruct((2 * HALF, STATE_SIZE), jnp.float32),
            jax.ShapeDtypeStruct((2 * HALF,), i32),
        ),
        mesh=mesh,
        scratch_types=[
            pltpu.VMEM((KB,), i32),
            pltpu.VMEM((KB,), i32),
            pltpu.VMEM((KB,), i32),
            pltpu.VMEM((KB, STATE_SIZE), jnp.float32),
            pltpu.SemaphoreType.DMA,
        ],
    )(_sc_gather_body)

    out_samples, out_counts = sc_gather(
        jnp.asarray(_POS_TAB), sorted_ids, uniq_count, uniq_samples)
    return out_samples, out_counts


# speedup vs baseline: 1.8239x; 1.8239x over previous
import functools

import jax
import jax.numpy as jnp
import numpy as np
from jax import lax
from jax.experimental import pallas as pl
from jax.experimental.pallas import tpu as pltpu
from jax.experimental.pallas import tpu_sc as plsc

N_UNIQ = 100000
STATE_SIZE = 128
HALF = 8192
NUM_LEFT = N_UNIQ - HALF

V = 1024
C = 200
B = 512
N_PAD = B * C
PAD_VAL = 1023

NW = 32
LANES = 16
LROW = 128
CPW = LANES
EPT = CPW * C
KB = (2 * HALF) // NW


def _np_threefry2x32(k1, k2, x0, x1):
    def rotl(x, d):
        return ((x << np.uint32(d)) | (x >> np.uint32(32 - d))).astype(np.uint32)
    ks = [np.uint32(k1), np.uint32(k2),
          np.uint32(k1) ^ np.uint32(k2) ^ np.uint32(0x1BD11BDA)]
    rots = [(13, 15, 26, 6), (17, 29, 16, 24)]
    x0 = (x0 + ks[0]).astype(np.uint32)
    x1 = (x1 + ks[1]).astype(np.uint32)
    for i in range(5):
        for r in rots[i % 2]:
            x0 = (x0 + x1).astype(np.uint32)
            x1 = rotl(x1, r)
            x1 = x0 ^ x1
        x0 = (x0 + ks[(i + 1) % 3]).astype(np.uint32)
        x1 = (x1 + ks[(i + 2) % 3] + np.uint32(i + 1)).astype(np.uint32)
    return x0, x1


def _np_permutation(seed, n):
    key = np.array([np.uint32(np.uint64(seed) >> np.uint64(32)),
                    np.uint32(np.uint64(seed) & np.uint64(0xFFFFFFFF))])
    x = np.arange(n, dtype=np.int32)
    num_rounds = int(np.ceil(3 * np.log(max(1, n)) / np.log(2**32 - 1)))
    for _ in range(num_rounds):
        b1, b2 = _np_threefry2x32(key[0], key[1],
                                  np.zeros(2, np.uint32),
                                  np.arange(2, dtype=np.uint32))
        key, subkey = np.stack([b1, b2], 1)
        s1, s2 = _np_threefry2x32(subkey[0], subkey[1],
                                  np.zeros(n, np.uint32),
                                  np.arange(n, dtype=np.uint32))
        x = x[np.argsort(s1 ^ s2, kind="stable")]
    return x


_PERM = _np_permutation(12345, NUM_LEFT)[:HALF]
_POS_TAB = np.concatenate([NUM_LEFT + np.arange(HALF, dtype=np.int32),
                           _PERM.astype(np.int32)])
_SELBITS = np.zeros(N_PAD // 32, np.int32)
np.bitwise_or.at(_SELBITS, _POS_TAB >> 5,
                 np.int32(1) << (_POS_TAB & 31))


def _sc_histrank_body(cnt_hbm, hist_out, rin_out, key_out,
                      cnt_v, hist_v, rin_v, key_v):
    wid = lax.axis_index("s") * 2 + lax.axis_index("c")
    base = wid * EPT
    pltpu.sync_copy(cnt_hbm.at[pl.ds(base, EPT)], cnt_v)
    iota16 = lax.broadcasted_iota(jnp.int32, (LANES,), 0)
    zeros16 = jnp.zeros((LANES,), jnp.int32)
    ones16 = jnp.ones((LANES,), jnp.int32)
    kbase = wid * CPW * V

    def zero_step(i, _):
        hist_v[pl.ds(i * LANES, LANES)] = zeros16
        return 0

    lax.fori_loop(0, CPW * V // LANES, zero_step, 0)

    def elem_step(e, _):
        idx16 = iota16 * C + e
        c16 = plsc.load_gather(cnt_v, [idx16])
        h_idx = iota16 * V + c16
        old = plsc.load_gather(hist_v, [h_idx])
        plsc.store_scatter(rin_v, [idx16], old)
        plsc.store_scatter(key_v, [idx16], h_idx + kbase)
        plsc.addupdate_scatter(hist_v, [h_idx], ones16)
        return 0

    lax.fori_loop(0, C, elem_step, 0)
    pltpu.sync_copy(hist_v, hist_out.at[pl.ds(wid * CPW * V, CPW * V)])
    pltpu.sync_copy(rin_v, rin_out.at[pl.ds(base, EPT)])
    pltpu.sync_copy(key_v, key_out.at[pl.ds(base, EPT)])


def _tc_offs_body(hist_ref, out_ref):
    hist = hist_ref[...].astype(jnp.float32)
    tri = (lax.broadcasted_iota(jnp.int32, (B, B), 1)
           < lax.broadcasted_iota(jnp.int32, (B, B), 0)).astype(jnp.float32)
    colcum = jnp.dot(tri, hist, preferred_element_type=jnp.float32)
    total = jnp.sum(hist, axis=0, keepdims=True)
    mv = (lax.broadcasted_iota(jnp.int32, (V, V), 0)
          < lax.broadcasted_iota(jnp.int32, (V, V), 1)).astype(jnp.float32)
    glob = jnp.dot(total, mv, preferred_element_type=jnp.float32)
    out_ref[...] = (colcum + glob).astype(jnp.int32)


def _sc_scatter_body(key_hbm, rin_hbm, choff_hbm, selb_hbm, sorted_out,
                     key_v, rin_v, choff_v, selb_v, ids_v, pos_v, sem):
    wid = lax.axis_index("s") * 2 + lax.axis_index("c")
    base = wid * EPT
    pltpu.sync_copy(key_hbm.at[pl.ds(base, EPT)], key_v)
    pltpu.sync_copy(rin_hbm.at[pl.ds(base, EPT)], rin_v)
    pltpu.sync_copy(choff_hbm.at[pl.ds(wid * CPW * V, CPW * V)], choff_v)
    pltpu.sync_copy(selb_hbm, selb_v)
    iota16 = lax.broadcasted_iota(jnp.int32, (LANES,), 0)
    kbase = wid * CPW * V
    gpr = LROW // LANES

    def pos_step(g, _):
        k16 = key_v[pl.ds(g * LANES, LANES)] - kbase
        off16 = plsc.load_gather(choff_v, [k16])
        p16 = off16 + rin_v[pl.ds(g * LANES, LANES)]
        w16 = plsc.load_gather(selb_v, [lax.shift_right_logical(p16, 5)])
        bit = lax.shift_right_logical(w16, p16 & 31) & 1
        trash16 = N_PAD + base + g * LANES + iota16
        pos_v[g // gpr, pl.ds((g % gpr) * LANES, LANES)] = jnp.where(
            bit != 0, p16, trash16)
        ids_v[pl.ds(g * LANES, LANES)] = base + g * LANES + iota16
        return 0

    lax.fori_loop(0, EPT // LANES, pos_step, 0)
    cps = [pltpu.async_copy(ids_v.at[pl.ds(j * LROW, LROW)],
                            sorted_out.at[pos_v.at[j]], sem)
           for j in range(EPT // LROW)]
    for cp in cps:
        cp.wait()


def _sc_gather_body(ptab_hbm, sorted_hbm, cnt_hbm, samp_hbm,
                    samp_out, cnt_out, ptab_v, id_v, cntg_v, rows_v, sem):
    wid = lax.axis_index("s") * 2 + lax.axis_index("c")
    base = wid * KB
    pltpu.sync_copy(ptab_hbm.at[pl.ds(base, KB)], ptab_v)
    cps = [pltpu.async_copy(sorted_hbm.at[ptab_v.at[pl.ds(j * LROW, LROW)]],
                            id_v.at[pl.ds(j * LROW, LROW)], sem)
           for j in range(KB // LROW)]
    for cp in cps:
        cp.wait()
    cps = [pltpu.async_copy(cnt_hbm.at[id_v.at[pl.ds(j * LROW, LROW)]],
                            cntg_v.at[pl.ds(j * LROW, LROW)], sem)
           for j in range(KB // LROW)]
    cps += [pltpu.async_copy(samp_hbm.at[id_v.at[pl.ds(j * LROW, LROW)]],
                             rows_v.at[pl.ds(j * LROW, LROW)], sem)
            for j in range(KB // LROW)]
    for cp in cps:
        cp.wait()
    pltpu.sync_copy(cntg_v, cnt_out.at[pl.ds(base, KB)])
    pltpu.sync_copy(rows_v, samp_out.at[pl.ds(base, KB)])


def kernel(uniq_samples, uniq_count):
    i32 = jnp.int32
    cpad = jnp.concatenate(
        [uniq_count, jnp.full((N_PAD - N_UNIQ,), PAD_VAL, dtype=i32)])

    mesh = plsc.VectorSubcoreMesh(
        core_axis_name="c", subcore_axis_name="s",
        num_cores=2, num_subcores=16)

    sc_histrank = functools.partial(
        pl.kernel,
        out_type=(
            jax.ShapeDtypeStruct((B * V,), i32),
            jax.ShapeDtypeStruct((N_PAD,), i32),
            jax.ShapeDtypeStruct((N_PAD,), i32),
        ),
        mesh=mesh,
        scratch_types=[pltpu.VMEM((EPT,), i32),
                       pltpu.VMEM((CPW * V,), i32),
                       pltpu.VMEM((EPT,), i32),
                       pltpu.VMEM((EPT,), i32)],
        compiler_params=pltpu.CompilerParams(needs_layout_passes=False),
    )(_sc_histrank_body)

    hist, rin, key = sc_histrank(cpad)

    choff = pl.pallas_call(
        _tc_offs_body,
        out_shape=jax.ShapeDtypeStruct((B, V), i32),
    )(hist.reshape(B, V))

    sc_scatter = functools.partial(
        pl.kernel,
        out_type=jax.ShapeDtypeStruct((2 * N_PAD,), i32),
        mesh=mesh,
        scratch_types=[
            pltpu.VMEM((EPT,), i32),
            pltpu.VMEM((EPT,), i32),
            pltpu.VMEM((CPW * V,), i32),
            pltpu.VMEM((N_PAD // 32,), i32),
            pltpu.VMEM((EPT,), i32),
            pltpu.VMEM((EPT // LROW, LROW), i32),
            pltpu.SemaphoreType.DMA,
        ],
        compiler_params=pltpu.CompilerParams(needs_layout_passes=False),
    )(_sc_scatter_body)

    sorted_ids = sc_scatter(key, rin, choff.reshape(-1), jnp.asarray(_SELBITS))

    sc_gather = functools.partial(
        pl.kernel,
        out_type=(
            jax.ShapeDtypeStruct((2 * HALF, STATE_SIZE), jnp.float32),
            jax.ShapeDtypeStruct((2 * HALF,), i32),
        ),
        mesh=mesh,
        scratch_types=[
            pltpu.VMEM((KB,), i32),
            pltpu.VMEM((KB,), i32),
            pltpu.VMEM((KB,), i32),
            pltpu.VMEM((KB, STATE_SIZE), jnp.float32),
            pltpu.SemaphoreType.DMA,
        ],
    )(_sc_gather_body)

    out_samples, out_counts = sc_gather(
        jnp.asarray(_POS_TAB), sorted_ids, uniq_count, uniq_samples)
    return out_samples, out_counts


# speedup vs baseline: 4.8688x; 2.6694x over previous
import functools

import jax
import jax.numpy as jnp
import numpy as np
from jax import lax
from jax.experimental import pallas as pl
from jax.experimental.pallas import tpu as pltpu
from jax.experimental.pallas import tpu_sc as plsc

N_UNIQ = 100000
STATE_SIZE = 128
HALF = 8192
NUM_LEFT = N_UNIQ - HALF

V = 1024
C = 200
B = 512
N_PAD = B * C
PAD_VAL = 1023

NW = 32
LANES = 16
LROW = 128
CPW = LANES
EPT = CPW * C
KB = (2 * HALF) // NW


def _np_threefry2x32(k1, k2, x0, x1):
    def rotl(x, d):
        return ((x << np.uint32(d)) | (x >> np.uint32(32 - d))).astype(np.uint32)
    ks = [np.uint32(k1), np.uint32(k2),
          np.uint32(k1) ^ np.uint32(k2) ^ np.uint32(0x1BD11BDA)]
    rots = [(13, 15, 26, 6), (17, 29, 16, 24)]
    x0 = (x0 + ks[0]).astype(np.uint32)
    x1 = (x1 + ks[1]).astype(np.uint32)
    for i in range(5):
        for r in rots[i % 2]:
            x0 = (x0 + x1).astype(np.uint32)
            x1 = rotl(x1, r)
            x1 = x0 ^ x1
        x0 = (x0 + ks[(i + 1) % 3]).astype(np.uint32)
        x1 = (x1 + ks[(i + 2) % 3] + np.uint32(i + 1)).astype(np.uint32)
    return x0, x1


def _np_permutation(seed, n):
    key = np.array([np.uint32(np.uint64(seed) >> np.uint64(32)),
                    np.uint32(np.uint64(seed) & np.uint64(0xFFFFFFFF))])
    x = np.arange(n, dtype=np.int32)
    num_rounds = int(np.ceil(3 * np.log(max(1, n)) / np.log(2**32 - 1)))
    for _ in range(num_rounds):
        b1, b2 = _np_threefry2x32(key[0], key[1],
                                  np.zeros(2, np.uint32),
                                  np.arange(2, dtype=np.uint32))
        key, subkey = np.stack([b1, b2], 1)
        s1, s2 = _np_threefry2x32(subkey[0], subkey[1],
                                  np.zeros(n, np.uint32),
                                  np.arange(n, dtype=np.uint32))
        x = x[np.argsort(s1 ^ s2, kind="stable")]
    return x


_PERM = _np_permutation(12345, NUM_LEFT)[:HALF]
_POS_TAB = np.concatenate([NUM_LEFT + np.arange(HALF, dtype=np.int32),
                           _PERM.astype(np.int32)])
_SELBITS = np.zeros(N_PAD // 32, np.int32)
np.bitwise_or.at(_SELBITS, _POS_TAB >> 5,
                 np.int32(1) << (_POS_TAB & 31))


def _sc_histrank_body(cnt_hbm, hist_out, rin_out, key_out,
                      cnt_v, hist_v, rin_v, key_v):
    wid = lax.axis_index("s") * 2 + lax.axis_index("c")
    base = wid * EPT
    pltpu.sync_copy(cnt_hbm.at[pl.ds(base, EPT)], cnt_v)
    iota16 = lax.broadcasted_iota(jnp.int32, (LANES,), 0)
    zeros16 = jnp.zeros((LANES,), jnp.int32)
    ones16 = jnp.ones((LANES,), jnp.int32)
    kbase = wid * CPW * V

    def zero_step(i, _):
        hist_v[pl.ds(i * LANES, LANES)] = zeros16
        return 0

    lax.fori_loop(0, CPW * V // LANES, zero_step, 0)

    def elem_step(e, _):
        idx16 = iota16 * C + e
        c16 = plsc.load_gather(cnt_v, [idx16])
        h_idx = iota16 * V + c16
        old = plsc.load_gather(hist_v, [h_idx])
        plsc.store_scatter(rin_v, [idx16], old)
        plsc.store_scatter(key_v, [idx16], h_idx + kbase)
        plsc.addupdate_scatter(hist_v, [h_idx], ones16)
        return 0

    lax.fori_loop(0, C, elem_step, 0)
    pltpu.sync_copy(hist_v, hist_out.at[pl.ds(wid * CPW * V, CPW * V)])
    pltpu.sync_copy(rin_v, rin_out.at[pl.ds(base, EPT)])
    pltpu.sync_copy(key_v, key_out.at[pl.ds(base, EPT)])


def _tc_offs_body(hist_ref, out_ref):
    hist = hist_ref[...].astype(jnp.float32)
    tri = (lax.broadcasted_iota(jnp.int32, (B, B), 1)
           < lax.broadcasted_iota(jnp.int32, (B, B), 0)).astype(jnp.float32)
    colcum = jnp.dot(tri, hist, preferred_element_type=jnp.float32)
    total = jnp.sum(hist, axis=0, keepdims=True)
    mv = (lax.broadcasted_iota(jnp.int32, (V, V), 0)
          < lax.broadcasted_iota(jnp.int32, (V, V), 1)).astype(jnp.float32)
    glob = jnp.dot(total, mv, preferred_element_type=jnp.float32)
    out_ref[...] = (colcum + glob).astype(jnp.int32)


def _sc_scatter_body(key_hbm, rin_hbm, choff_hbm, selb_hbm, sorted_out,
                     key_v, rin_v, choff_v, selb_v, cids_v, cpos_v, pos_v,
                     sem):
    wid = lax.axis_index("s") * 2 + lax.axis_index("c")
    base = wid * EPT
    pltpu.sync_copy(key_hbm.at[pl.ds(base, EPT)], key_v)
    pltpu.sync_copy(rin_hbm.at[pl.ds(base, EPT)], rin_v)
    pltpu.sync_copy(choff_hbm.at[pl.ds(wid * CPW * V, CPW * V)], choff_v)
    pltpu.sync_copy(selb_hbm, selb_v)
    iota16 = lax.broadcasted_iota(jnp.int32, (LANES,), 0)
    kbase = wid * CPW * V
    gpr = LROW // LANES

    def fill_step(g, _):
        cpos_v[pl.ds(g * LANES, LANES)] = N_PAD + base + g * LANES + iota16
        return 0

    lax.fori_loop(0, EPT // LANES, fill_step, 0)

    def pos_step(g, ptr):
        k16 = key_v[pl.ds(g * LANES, LANES)] - kbase
        off16 = plsc.load_gather(choff_v, [k16])
        p16 = off16 + rin_v[pl.ds(g * LANES, LANES)]
        w16 = plsc.load_gather(selb_v, [lax.shift_right_logical(p16, 5)])
        sel = (lax.shift_right_logical(w16, p16 & 31) & 1) != 0
        plsc.store_compressed(cpos_v.at[pl.ds(ptr, LANES)], p16, mask=sel)
        plsc.store_compressed(cids_v.at[pl.ds(ptr, LANES)],
                              base + g * LANES + iota16, mask=sel)
        return ptr + jnp.sum(sel.astype(jnp.int32))

    nsel = lax.fori_loop(0, EPT // LANES, pos_step, jnp.int32(0))
    nrow = (nsel + (LROW - 1)) // LROW

    def expand_step(g, _):
        pos_v[g // gpr, pl.ds((g % gpr) * LANES, LANES)] = (
            cpos_v[pl.ds(g * LANES, LANES)])
        return 0

    lax.fori_loop(0, nrow * gpr, expand_step, 0)

    def issue_step(j, _):
        pltpu.async_copy(cids_v.at[pl.ds(j * LROW, LROW)],
                         sorted_out.at[pos_v.at[j]], sem)
        return 0

    lax.fori_loop(0, nrow, issue_step, 0)

    def drain_step(j, _):
        pltpu.make_async_copy(cids_v.at[pl.ds(0, LROW)],
                              sorted_out.at[pos_v.at[0]], sem).wait()
        return 0

    lax.fori_loop(0, nrow, drain_step, 0)


def _sc_gather_body(ptab_hbm, sorted_hbm, cnt_hbm, samp_hbm,
                    samp_out, cnt_out, ptab_v, id_v, cntg_v, rows_v, sem):
    wid = lax.axis_index("s") * 2 + lax.axis_index("c")
    base = wid * KB
    pltpu.sync_copy(ptab_hbm.at[pl.ds(base, KB)], ptab_v)
    cps = [pltpu.async_copy(sorted_hbm.at[ptab_v.at[pl.ds(j * LROW, LROW)]],
                            id_v.at[pl.ds(j * LROW, LROW)], sem)
           for j in range(KB // LROW)]
    for cp in cps:
        cp.wait()
    cps = [pltpu.async_copy(cnt_hbm.at[id_v.at[pl.ds(j * LROW, LROW)]],
                            cntg_v.at[pl.ds(j * LROW, LROW)], sem)
           for j in range(KB // LROW)]
    cps += [pltpu.async_copy(samp_hbm.at[id_v.at[pl.ds(j * LROW, LROW)]],
                             rows_v.at[pl.ds(j * LROW, LROW)], sem)
            for j in range(KB // LROW)]
    for cp in cps:
        cp.wait()
    pltpu.sync_copy(cntg_v, cnt_out.at[pl.ds(base, KB)])
    pltpu.sync_copy(rows_v, samp_out.at[pl.ds(base, KB)])


def kernel(uniq_samples, uniq_count):
    i32 = jnp.int32
    cpad = jnp.concatenate(
        [uniq_count, jnp.full((N_PAD - N_UNIQ,), PAD_VAL, dtype=i32)])

    mesh = plsc.VectorSubcoreMesh(
        core_axis_name="c", subcore_axis_name="s",
        num_cores=2, num_subcores=16)

    sc_histrank = functools.partial(
        pl.kernel,
        out_type=(
            jax.ShapeDtypeStruct((B * V,), i32),
            jax.ShapeDtypeStruct((N_PAD,), i32),
            jax.ShapeDtypeStruct((N_PAD,), i32),
        ),
        mesh=mesh,
        scratch_types=[pltpu.VMEM((EPT,), i32),
                       pltpu.VMEM((CPW * V,), i32),
                       pltpu.VMEM((EPT,), i32),
                       pltpu.VMEM((EPT,), i32)],
        compiler_params=pltpu.CompilerParams(needs_layout_passes=False),
    )(_sc_histrank_body)

    hist, rin, key = sc_histrank(cpad)

    choff = pl.pallas_call(
        _tc_offs_body,
        out_shape=jax.ShapeDtypeStruct((B, V), i32),
    )(hist.reshape(B, V))

    sc_scatter = functools.partial(
        pl.kernel,
        out_type=jax.ShapeDtypeStruct((2 * N_PAD,), i32),
        mesh=mesh,
        scratch_types=[
            pltpu.VMEM((EPT,), i32),
            pltpu.VMEM((EPT,), i32),
            pltpu.VMEM((CPW * V,), i32),
            pltpu.VMEM((N_PAD // 32,), i32),
            pltpu.VMEM((EPT,), i32),
            pltpu.VMEM((EPT,), i32),
            pltpu.VMEM((EPT // LROW, LROW), i32),
            pltpu.SemaphoreType.DMA,
        ],
        compiler_params=pltpu.CompilerParams(needs_layout_passes=False),
    )(_sc_scatter_body)

    sorted_ids = sc_scatter(key, rin, choff.reshape(-1), jnp.asarray(_SELBITS))

    sc_gather = functools.partial(
        pl.kernel,
        out_type=(
            jax.ShapeDtypeStruct((2 * HALF, STATE_SIZE), jnp.float32),
            jax.ShapeDtypeStruct((2 * HALF,), i32),
        ),
        mesh=mesh,
        scratch_types=[
            pltpu.VMEM((KB,), i32),
            pltpu.VMEM((KB,), i32),
            pltpu.VMEM((KB,), i32),
            pltpu.VMEM((KB, STATE_SIZE), jnp.float32),
            pltpu.SemaphoreType.DMA,
        ],
    )(_sc_gather_body)

    out_samples, out_counts = sc_gather(
        jnp.asarray(_POS_TAB), sorted_ids, uniq_count, uniq_samples)
    return out_samples, out_counts
